# baseline (speedup 1.0000x reference)
import functools

import jax
import jax.numpy as jnp
from jax import lax
from jax.experimental import pallas as pl
from jax.experimental.pallas import tpu as pltpu
from jax.experimental.pallas import tpu_sc as plsc

_NC = 2
_NS = 16
_NW = _NC * _NS


def kernel(patches):
    batch = patches.shape[0]
    G = batch * 8
    gpw = G // _NW

    in4 = patches.reshape(G, 8, 64, 192)

    mesh = plsc.VectorSubcoreMesh(core_axis_name="c", subcore_axis_name="s")

    @functools.partial(
        pl.kernel,
        mesh=mesh,
        out_type=jax.ShapeDtypeStruct((G * 64, 8 * 192), jnp.float32),
        scratch_types=[pltpu.SemaphoreType.DMA],
        compiler_params=pltpu.CompilerParams(use_tc_tiling_on_sc=False),
    )
    def unpatch(in_hbm, out_hbm, sem):
        wid = lax.axis_index("s") * _NC + lax.axis_index("c")
        g0 = wid * gpw

        copies = []
        for t in range(gpw):
            g = g0 + t
            row0 = g * 64
            for i in range(8):
                copies.append(
                    pltpu.async_copy(
                        in_hbm.at[g, i],
                        out_hbm.at[pl.ds(row0, 64), pl.ds(i * 192, 192)],
                        sem,
                    )
                )
        for cp in copies:
            cp.wait()

    out = unpatch(in4)
    return out.reshape(batch, 512, 512, 3)


# speedup vs baseline: 4.6664x; 4.6664x over previous
import functools

import jax
import jax.numpy as jnp
from jax import lax
from jax.experimental import pallas as pl
from jax.experimental.pallas import tpu as pltpu
from jax.experimental.pallas import tpu_sc as plsc

_NC = 2
_NS = 16
_NW = _NC * _NS


def kernel(patches):
    batch = patches.shape[0]
    G = batch * 8
    gpw = G // _NW

    in4 = patches.reshape(G, 8, 64, 192)

    mesh = plsc.VectorSubcoreMesh(core_axis_name="c", subcore_axis_name="s")

    @functools.partial(
        pl.kernel,
        mesh=mesh,
        out_type=jax.ShapeDtypeStruct((G * 64, 8 * 192), jnp.float32),
        scratch_types=[
            pltpu.VMEM((4, 64, 192), jnp.float32),
            pltpu.VMEM((4, 64, 192), jnp.float32),
            pltpu.SemaphoreType.DMA,
            pltpu.SemaphoreType.DMA,
            pltpu.SemaphoreType.DMA,
            pltpu.SemaphoreType.DMA,
        ],
        compiler_params=pltpu.CompilerParams(use_tc_tiling_on_sc=False),
    )
    def unpatch(in_hbm, out_hbm, buf_a, buf_b, sin_a, sin_b, sout_a, sout_b):
        wid = lax.axis_index("s") * _NC + lax.axis_index("c")
        g0 = wid * gpw

        bufs = [buf_a, buf_b]
        sins = [sin_a, sin_b]
        souts = [sout_a, sout_b]
        T = gpw * 2

        def load(t):
            g = g0 + t // 2
            h = t % 2
            return pltpu.async_copy(
                in_hbm.at[g, pl.ds(h * 4, 4)], bufs[t % 2], sins[t % 2]
            )

        def store(t):
            g = g0 + t // 2
            h = t % 2
            row0 = g * 64
            return [
                pltpu.async_copy(
                    bufs[t % 2].at[ii],
                    out_hbm.at[pl.ds(row0, 64), pl.ds((h * 4 + ii) * 192, 192)],
                    souts[t % 2],
                )
                for ii in range(4)
            ]

        in_cp = load(0)
        out_cps = [None] * T
        for t in range(T):
            in_cp.wait()
            out_cps[t] = store(t)
            if t + 1 < T:
                if t >= 1:
                    for cp in out_cps[t - 1]:
                        cp.wait()
                in_cp = load(t + 1)
        for cp in out_cps[T - 2]:
            cp.wait()
        for cp in out_cps[T - 1]:
            cp.wait()

    out = unpatch(in4)
    return out.reshape(batch, 512, 512, 3)


# speedup vs baseline: 20.6963x; 4.4352x over previous
import functools

import jax
import jax.numpy as jnp
from jax import lax
from jax.experimental import pallas as pl
from jax.experimental.pallas import tpu as pltpu
from jax.experimental.pallas import tpu_sc as plsc

_NC = 2
_NS = 16
_NW = _NC * _NS


def kernel(patches):
    batch = patches.shape[0]
    x = jnp.transpose(patches, (0, 1, 2, 5, 3, 4))

    n_items = batch * 8 * 4 * 2
    ipw = n_items // _NW

    mesh = plsc.VectorSubcoreMesh(core_axis_name="c", subcore_axis_name="s")

    @functools.partial(
        pl.kernel,
        mesh=mesh,
        out_type=jax.ShapeDtypeStruct((batch, 3, 512, 512), jnp.float32),
        scratch_types=[
            pltpu.VMEM((3, 32, 64), jnp.float32),
            pltpu.VMEM((3, 32, 64), jnp.float32),
            pltpu.VMEM((3, 32, 128), jnp.float32),
            pltpu.VMEM((3, 32, 64), jnp.float32),
            pltpu.VMEM((3, 32, 64), jnp.float32),
            pltpu.VMEM((3, 32, 128), jnp.float32),
            pltpu.SemaphoreType.DMA,
            pltpu.SemaphoreType.DMA,
            pltpu.SemaphoreType.DMA,
            pltpu.SemaphoreType.DMA,
        ],
    )
    def unpatch(in_hbm, out_hbm,
                b0a, b1a, lin_a, b0b, b1b, lin_b,
                sin_a, sin_b, sout_a, sout_b):
        wid = lax.axis_index("s") * _NC + lax.axis_index("c")
        t0 = wid * ipw

        b0s, b1s = [b0a, b0b], [b1a, b1b]
        lins = [lin_a, lin_b]
        sins = [sin_a, sin_b]
        souts = [sout_a, sout_b]

        def coords(k):
            t = t0 + k
            b = t // 64
            r = t % 64
            j = r // 8
            q = r % 8
            return b, j, q // 2, q % 2

        def load(k):
            b, j, i2, ph = coords(k)
            return [
                pltpu.async_copy(
                    in_hbm.at[b, j, 2 * i2 + h, :, pl.ds(ph * 32, 32), :],
                    (b0s if h == 0 else b1s)[k % 2],
                    sins[k % 2],
                )
                for h in range(2)
            ]

        def merge(k):
            b0, b1, lin = b0s[k % 2], b1s[k % 2], lins[k % 2]

            def body(py, carry):
                for c in range(3):
                    for kk in range(4):
                        sl = pl.ds(kk * 16, 16)
                        lin[c, py, sl] = b0[c, py, sl]
                        lin[c, py, pl.ds(64 + kk * 16, 16)] = b1[c, py, sl]
                return carry

            lax.fori_loop(0, 32, body, 0)

        def store(k):
            b, j, i2, ph = coords(k)
            return [
                pltpu.async_copy(
                    lins[k % 2],
                    out_hbm.at[b, :, pl.ds(j * 64 + ph * 32, 32),
                               pl.ds(i2 * 128, 128)],
                    souts[k % 2],
                )
            ]

        in_cps = load(0)
        out_cps = [None] * ipw
        for k in range(ipw):
            for cp in in_cps:
                cp.wait()
            if k + 1 < ipw:
                if k >= 1:
                    for cp in out_cps[k - 1]:
                        cp.wait()
                in_cps = load(k + 1)
            merge(k)
            out_cps[k] = store(k)
        for cp in out_cps[ipw - 2]:
            cp.wait()
        for cp in out_cps[ipw - 1]:
            cp.wait()

    out = unpatch(x)
    return jnp.transpose(out, (0, 2, 3, 1))
